# baseline (speedup 1.0000x reference)
import functools

import jax
import jax.numpy as jnp
from jax import lax
from jax.experimental import pallas as pl
from jax.experimental.pallas import tpu as pltpu
from jax.experimental.pallas import tpu_sc as plsc

_N = 10000
_BATCH = 100
_E = 320000
_D = 128
_NPAD = 10240

_NC = 2
_NS = 16
_NW = _NC * _NS
_CH = 80
_CHW = 128
_EPAD = _CHW * _CH * _NW


def _make_agg():
  mesh = plsc.VectorSubcoreMesh(core_axis_name="c", subcore_axis_name="s")
  rows_per_tile = _NPAD // _NS
  _ZR = 32

  @functools.partial(
      pl.kernel,
      mesh=mesh,
      out_type=jax.ShapeDtypeStruct((_NC, _NPAD, _D), jnp.float32),
      scratch_types=(
          [pltpu.VMEM((_CH,), jnp.int32) for _ in range(16)]
          + [pltpu.VMEM((_CH, _D), jnp.float32) for _ in range(4)]
          + [pltpu.VMEM((_ZR, _D), jnp.float32)]
          + [pltpu.VMEM_SHARED((_NPAD, _D), jnp.float32)]
          + [pltpu.SemaphoreType.DMA for _ in range(17)]
      ),
  )
  def agg(feats_hbm, src_hbm, dst_hbm, out_hbm,
          sv0, sv1, sv2, sv3, sv4, sv5, sv6, sv7,
          dv0, dv1, dv2, dv3, dv4, dv5, dv6, dv7,
          rows0, rows1, rows2, rows3, zero_v, acc_sh,
          si0, si1, si2, si3, si4, si5, si6, si7,
          sg0, sg1, sg2, sg3, ss0, ss1, ss2, ss3, sz):
    cid = lax.axis_index("c")
    sid = lax.axis_index("s")
    wid = sid * _NC + cid

    srcs = (sv0, sv1, sv2, sv3, sv4, sv5, sv6, sv7)
    dsts = (dv0, dv1, dv2, dv3, dv4, dv5, dv6, dv7)
    rows = (rows0, rows1, rows2, rows3)
    sem_i = (si0, si1, si2, si3, si4, si5, si6, si7)
    sem_g = (sg0, sg1, sg2, sg3)
    sem_s = (ss0, ss1, ss2, ss3)

    def chunk_off(s):
      return (wid + s * _NW) * _CH

    def issue_idx(b, off):
      pltpu.async_copy(src_hbm.at[pl.ds(off, _CH)], srcs[b], sem_i[b])
      pltpu.async_copy(dst_hbm.at[pl.ds(off, _CH)], dsts[b], sem_i[b])

    def wait_idx(b):
      pltpu.make_async_copy(
          src_hbm.at[pl.ds(0, _CH)], srcs[b], sem_i[b]).wait()
      pltpu.make_async_copy(
          dst_hbm.at[pl.ds(0, _CH)], dsts[b], sem_i[b]).wait()

    def issue_gather(b, j):
      pltpu.async_copy(feats_hbm.at[srcs[j]], rows[b], sem_g[b])

    def wait_gather(b, j):
      pltpu.make_async_copy(
          feats_hbm.at[srcs[j]], rows[b], sem_g[b]).wait()

    def issue_scatter(b, j):
      pltpu.async_copy(rows[b], acc_sh.at[dsts[j]], sem_s[b], add=True)

    def wait_scatter(b, j):
      pltpu.make_async_copy(
          rows[b], acc_sh.at[dsts[j]], sem_s[b]).wait()

    for j in range(4):
      issue_idx(j, chunk_off(j))

    z16 = jnp.zeros((16,), jnp.float32)
    for r in range(_ZR):
      for c in range(_D // 16):
        zero_v[r, pl.ds(c * 16, 16)] = z16
    for i in range(rows_per_tile // _ZR):
      pltpu.async_copy(
          zero_v, acc_sh.at[pl.ds(sid * rows_per_tile + i * _ZR, _ZR)], sz)
    for i in range(rows_per_tile // _ZR):
      pltpu.make_async_copy(
          zero_v, acc_sh.at[pl.ds(sid * rows_per_tile, _ZR)], sz).wait()
    plsc.subcore_barrier()

    def half(s, b, j, first=False, last=False):
      bn, jn = (b + 1) % 4, (j + 1) % 8
      wait_gather(b, j)
      if not first:
        wait_scatter(bn, (j + 5) % 8)
      if not last:
        wait_idx(jn)
        issue_gather(bn, jn)
      issue_scatter(b, j)
      return None

    wait_idx(0)
    issue_gather(0, 0)

    for s in range(3):
      half(s, s % 4, s % 8, first=True)
      issue_idx(s + 4, chunk_off(s + 4))
    half(3, 3, 3)
    issue_idx(7, chunk_off(7))

    def octet(k, carry):
      s0 = 8 * k + 4
      for d in range(8):
        s = s0 + d
        half(s, d % 4, (4 + d) % 8)
        issue_idx(d % 8, chunk_off(s + 4))
      return carry
    n_oct = (_CHW - 8) // 8
    lax.fori_loop(0, n_oct, octet, 0)

    for s in range(8 * n_oct + 4, _CHW - 1):
      half(s, s % 4, s % 8)
    half(_CHW - 1, (_CHW - 1) % 4, (_CHW - 1) % 8, last=True)

    for s in range(_CHW - 3, _CHW):
      wait_scatter(s % 4, s % 8)

    plsc.subcore_barrier()

    pltpu.sync_copy(
        acc_sh.at[pl.ds(sid * rows_per_tile, rows_per_tile)],
        out_hbm.at[cid, pl.ds(sid * rows_per_tile, rows_per_tile)])

  return agg


_agg = _make_agg()

_R = 1000


def _gru(parts, feats, wl_t, bl, wi_t, wh_t, bi, bh, w0_t=None):
  def body(p_ref, f_ref, wl_ref, bl_ref, wi_ref, wh_ref, bi_ref, bh_ref,
           *rest):
    o_ref = rest[-1]
    agg = p_ref[0] + p_ref[1]
    f = f_ref[...]
    if w0_t is not None:
      f = jnp.dot(f, rest[0][...], preferred_element_type=jnp.float32,
                  precision=jax.lax.Precision.HIGHEST)
      h = jnp.dot(agg, wl_ref[...], preferred_element_type=jnp.float32,
                  precision=jax.lax.Precision.HIGHEST) + bl_ref[...]
    else:
      h = jnp.dot(agg, wl_ref[...], preferred_element_type=jnp.float32) + bl_ref[...]
    gi = jnp.dot(h, wi_ref[...], preferred_element_type=jnp.float32) + bi_ref[...]
    gh = jnp.dot(f, wh_ref[...], preferred_element_type=jnp.float32) + bh_ref[...]
    r = jax.nn.sigmoid(gi[:, :_D] + gh[:, :_D])
    z = jax.nn.sigmoid(gi[:, _D:2 * _D] + gh[:, _D:2 * _D])
    n = jnp.tanh(gi[:, 2 * _D:] + r * gh[:, 2 * _D:])
    o_ref[...] = (1.0 - z) * n + z * f

  in_specs = [
      pl.BlockSpec((_NC, _R, _D), lambda i: (0, i, 0)),
      pl.BlockSpec((_R, _D), lambda i: (i, 0)),
      pl.BlockSpec((_D, _D), lambda i: (0, 0)),
      pl.BlockSpec((1, _D), lambda i: (0, 0)),
      pl.BlockSpec((_D, 3 * _D), lambda i: (0, 0)),
      pl.BlockSpec((_D, 3 * _D), lambda i: (0, 0)),
      pl.BlockSpec((1, 3 * _D), lambda i: (0, 0)),
      pl.BlockSpec((1, 3 * _D), lambda i: (0, 0)),
  ]
  args = [parts, feats, wl_t, bl, wi_t, wh_t, bi, bh]
  if w0_t is not None:
    in_specs.append(pl.BlockSpec((_D, _D), lambda i: (0, 0)))
    args.append(w0_t)
  return pl.pallas_call(
      body,
      grid=(_N // _R,),
      in_specs=in_specs,
      out_specs=pl.BlockSpec((_R, _D), lambda i: (i, 0)),
      out_shape=jax.ShapeDtypeStruct((_N, _D), jnp.float32),
  )(*args)


def kernel(x, edge_index, batch_size, W_init,
           W_lin0, b_lin0, W_ih0, W_hh0, b_ih0, b_hh0,
           W_lin1, b_lin1, W_ih1, W_hh1, b_ih1, b_hh1):
  src = edge_index[0].astype(jnp.int32)
  dst = edge_index[1].astype(jnp.int32)

  npadgap = _NPAD - _N
  pad = _EPAD - _E
  pad_i = jnp.arange(pad, dtype=jnp.int32)
  src_p = jnp.concatenate([src, pad_i % _N])
  dst_p = jnp.concatenate([dst, _N + pad_i % npadgap])

  wc0 = jnp.dot(W_init.T, W_lin0.T, precision=jax.lax.Precision.HIGHEST)
  parts = _agg(x, src_p, dst_p)
  feats = _gru(parts, x, wc0, b_lin0.reshape(1, -1),
               W_ih0.T, W_hh0.T, b_ih0.reshape(1, -1), b_hh0.reshape(1, -1),
               w0_t=W_init.T)

  parts = _agg(feats, src_p, dst_p)
  feats = _gru(parts, feats, W_lin1.T, b_lin1.reshape(1, -1),
               W_ih1.T, W_hh1.T, b_ih1.reshape(1, -1), b_hh1.reshape(1, -1))

  return feats.reshape(_BATCH, -1, _D)


# speedup vs baseline: 1.0309x; 1.0309x over previous
import functools

import jax
import jax.numpy as jnp
from jax import lax
from jax.experimental import pallas as pl
from jax.experimental.pallas import tpu as pltpu
from jax.experimental.pallas import tpu_sc as plsc

_N = 10000
_BATCH = 100
_E = 320000
_D = 128
_NPAD = 10240

_NC = 2
_NS = 16
_NW = _NC * _NS
_CH = 80
_CHW = 128
_EPAD = _CHW * _CH * _NW


def _make_agg():
  mesh = plsc.VectorSubcoreMesh(core_axis_name="c", subcore_axis_name="s")
  rows_per_tile = _NPAD // _NS
  _ZR = 32

  @functools.partial(
      pl.kernel,
      mesh=mesh,
      out_type=jax.ShapeDtypeStruct((_NC, _NPAD, _D), jnp.float32),
      scratch_types=(
          [pltpu.VMEM((_CH,), jnp.int32) for _ in range(16)]
          + [pltpu.VMEM((_CH, _D), jnp.float32) for _ in range(4)]
          + [pltpu.VMEM((_ZR, _D), jnp.float32)]
          + [pltpu.VMEM_SHARED((_NPAD, _D), jnp.float32)]
          + [pltpu.SemaphoreType.DMA for _ in range(17)]
      ),
  )
  def agg(feats_hbm, src_hbm, dst_hbm, out_hbm,
          sv0, sv1, sv2, sv3, sv4, sv5, sv6, sv7,
          dv0, dv1, dv2, dv3, dv4, dv5, dv6, dv7,
          rows0, rows1, rows2, rows3, zero_v, acc_sh,
          si0, si1, si2, si3, si4, si5, si6, si7,
          sg0, sg1, sg2, sg3, ss0, ss1, ss2, ss3, sz):
    cid = lax.axis_index("c")
    sid = lax.axis_index("s")
    wid = sid * _NC + cid

    srcs = (sv0, sv1, sv2, sv3, sv4, sv5, sv6, sv7)
    dsts = (dv0, dv1, dv2, dv3, dv4, dv5, dv6, dv7)
    rows = (rows0, rows1, rows2, rows3)
    sem_i = (si0, si1, si2, si3, si4, si5, si6, si7)
    sem_g = (sg0, sg1, sg2, sg3)
    sem_s = (ss0, ss1, ss2, ss3)

    def chunk_off(s):
      return (wid + s * _NW) * _CH

    def issue_idx(b, off):
      pltpu.async_copy(src_hbm.at[pl.ds(off, _CH)], srcs[b], sem_i[b])
      pltpu.async_copy(dst_hbm.at[pl.ds(off, _CH)], dsts[b], sem_i[b])

    def wait_idx(b):
      pltpu.make_async_copy(
          src_hbm.at[pl.ds(0, _CH)], srcs[b], sem_i[b]).wait()
      pltpu.make_async_copy(
          dst_hbm.at[pl.ds(0, _CH)], dsts[b], sem_i[b]).wait()

    def issue_gather(b, j):
      pltpu.async_copy(feats_hbm.at[srcs[j]], rows[b], sem_g[b])

    def wait_gather(b, j):
      pltpu.make_async_copy(
          feats_hbm.at[srcs[j]], rows[b], sem_g[b]).wait()

    def issue_scatter(b, j):
      pltpu.async_copy(rows[b], acc_sh.at[dsts[j]], sem_s[b], add=True)

    def wait_scatter(b, j):
      pltpu.make_async_copy(
          rows[b], acc_sh.at[dsts[j]], sem_s[b]).wait()

    for j in range(4):
      issue_idx(j, chunk_off(j))

    z16 = jnp.zeros((16,), jnp.float32)
    for r in range(_ZR):
      for c in range(_D // 16):
        zero_v[r, pl.ds(c * 16, 16)] = z16
    for i in range(rows_per_tile // _ZR):
      pltpu.async_copy(
          zero_v, acc_sh.at[pl.ds(sid * rows_per_tile + i * _ZR, _ZR)], sz)
    for i in range(rows_per_tile // _ZR):
      pltpu.make_async_copy(
          zero_v, acc_sh.at[pl.ds(sid * rows_per_tile, _ZR)], sz).wait()
    plsc.subcore_barrier()

    def half(s, b, j, first=False, last=False):
      bn, jn = (b + 1) % 4, (j + 1) % 8
      wait_gather(b, j)
      if not first:
        wait_scatter(bn, (j + 5) % 8)
      if not last:
        wait_idx(jn)
        issue_gather(bn, jn)
      issue_scatter(b, j)
      return None

    wait_idx(0)
    issue_gather(0, 0)

    for s in range(3):
      half(s, s % 4, s % 8, first=True)
      issue_idx(s + 4, chunk_off(s + 4))
    half(3, 3, 3)
    issue_idx(7, chunk_off(7))

    def octet(k, carry):
      s0 = 8 * k + 4
      for d in range(8):
        s = s0 + d
        half(s, d % 4, (4 + d) % 8)
        issue_idx(d % 8, chunk_off(s + 4))
      return carry
    n_oct = (_CHW - 8) // 8
    lax.fori_loop(0, n_oct, octet, 0)

    for s in range(8 * n_oct + 4, _CHW - 1):
      half(s, s % 4, s % 8)
    half(_CHW - 1, (_CHW - 1) % 4, (_CHW - 1) % 8, last=True)

    for s in range(_CHW - 3, _CHW):
      wait_scatter(s % 4, s % 8)

    plsc.subcore_barrier()

    pltpu.sync_copy(
        acc_sh.at[pl.ds(sid * rows_per_tile, rows_per_tile)],
        out_hbm.at[cid, pl.ds(sid * rows_per_tile, rows_per_tile)])

  return agg


_agg = _make_agg()

_R = 1000


def _gru(parts, feats, wl_t, bl, wi_t, wh_t, bi, bh, w0_t=None):
  def body(p_ref, f_ref, wl_ref, bl_ref, wi_ref, wh_ref, bi_ref, bh_ref,
           *rest):
    o_ref = rest[-1]
    agg = p_ref[0] + p_ref[1]
    f = f_ref[...]
    if w0_t is not None:
      f = jnp.dot(f, rest[0][...], preferred_element_type=jnp.float32)
    h = jnp.dot(agg, wl_ref[...], preferred_element_type=jnp.float32) + bl_ref[...]
    gi = jnp.dot(h, wi_ref[...], preferred_element_type=jnp.float32) + bi_ref[...]
    gh = jnp.dot(f, wh_ref[...], preferred_element_type=jnp.float32) + bh_ref[...]
    r = jax.nn.sigmoid(gi[:, :_D] + gh[:, :_D])
    z = jax.nn.sigmoid(gi[:, _D:2 * _D] + gh[:, _D:2 * _D])
    n = jnp.tanh(gi[:, 2 * _D:] + r * gh[:, 2 * _D:])
    o_ref[...] = (1.0 - z) * n + z * f

  in_specs = [
      pl.BlockSpec((_NC, _R, _D), lambda i: (0, i, 0)),
      pl.BlockSpec((_R, _D), lambda i: (i, 0)),
      pl.BlockSpec((_D, _D), lambda i: (0, 0)),
      pl.BlockSpec((1, _D), lambda i: (0, 0)),
      pl.BlockSpec((_D, 3 * _D), lambda i: (0, 0)),
      pl.BlockSpec((_D, 3 * _D), lambda i: (0, 0)),
      pl.BlockSpec((1, 3 * _D), lambda i: (0, 0)),
      pl.BlockSpec((1, 3 * _D), lambda i: (0, 0)),
  ]
  args = [parts, feats, wl_t, bl, wi_t, wh_t, bi, bh]
  if w0_t is not None:
    in_specs.append(pl.BlockSpec((_D, _D), lambda i: (0, 0)))
    args.append(w0_t)
  return pl.pallas_call(
      body,
      grid=(_N // _R,),
      in_specs=in_specs,
      out_specs=pl.BlockSpec((_R, _D), lambda i: (i, 0)),
      out_shape=jax.ShapeDtypeStruct((_N, _D), jnp.float32),
  )(*args)


def kernel(x, edge_index, batch_size, W_init,
           W_lin0, b_lin0, W_ih0, W_hh0, b_ih0, b_hh0,
           W_lin1, b_lin1, W_ih1, W_hh1, b_ih1, b_hh1):
  src = edge_index[0].astype(jnp.int32)
  dst = edge_index[1].astype(jnp.int32)

  npadgap = _NPAD - _N
  pad = _EPAD - _E
  pad_i = jnp.arange(pad, dtype=jnp.int32)
  src_p = jnp.concatenate([src, pad_i % _N])
  dst_p = jnp.concatenate([dst, _N + pad_i % npadgap])

  wc0 = jnp.dot(W_init.T, W_lin0.T, precision=jax.lax.Precision.HIGHEST)
  parts = _agg(x, src_p, dst_p)
  feats = _gru(parts, x, wc0, b_lin0.reshape(1, -1),
               W_ih0.T, W_hh0.T, b_ih0.reshape(1, -1), b_hh0.reshape(1, -1),
               w0_t=W_init.T)

  parts = _agg(feats, src_p, dst_p)
  feats = _gru(parts, feats, W_lin1.T, b_lin1.reshape(1, -1),
               W_ih1.T, W_hh1.T, b_ih1.reshape(1, -1), b_hh1.reshape(1, -1))

  return feats.reshape(_BATCH, -1, _D)


# speedup vs baseline: 1.0345x; 1.0034x over previous
import functools

import jax
import jax.numpy as jnp
from jax import lax
from jax.experimental import pallas as pl
from jax.experimental.pallas import tpu as pltpu
from jax.experimental.pallas import tpu_sc as plsc

_N = 10000
_BATCH = 100
_E = 320000
_D = 128
_NPAD = 10240

_NC = 2
_NS = 16
_NW = _NC * _NS
_CH = 80
_CHW = 128
_EPAD = _CHW * _CH * _NW


def _make_agg():
  mesh = plsc.VectorSubcoreMesh(core_axis_name="c", subcore_axis_name="s")
  rows_per_tile = _NPAD // _NS
  _ZR = 32

  @functools.partial(
      pl.kernel,
      mesh=mesh,
      out_type=jax.ShapeDtypeStruct((_NC, _NPAD, _D), jnp.float32),
      scratch_types=(
          [pltpu.VMEM((_CH,), jnp.int32) for _ in range(16)]
          + [pltpu.VMEM((_CH, _D), jnp.float32) for _ in range(4)]
          + [pltpu.VMEM((_ZR, _D), jnp.float32)]
          + [pltpu.VMEM_SHARED((_NPAD, _D), jnp.float32)]
          + [pltpu.SemaphoreType.DMA for _ in range(17)]
      ),
  )
  def agg(feats_hbm, src_hbm, dst_hbm, out_hbm,
          sv0, sv1, sv2, sv3, sv4, sv5, sv6, sv7,
          dv0, dv1, dv2, dv3, dv4, dv5, dv6, dv7,
          rows0, rows1, rows2, rows3, zero_v, acc_sh,
          si0, si1, si2, si3, si4, si5, si6, si7,
          sg0, sg1, sg2, sg3, ss0, ss1, ss2, ss3, sz):
    cid = lax.axis_index("c")
    sid = lax.axis_index("s")
    wid = sid * _NC + cid

    srcs = (sv0, sv1, sv2, sv3, sv4, sv5, sv6, sv7)
    dsts = (dv0, dv1, dv2, dv3, dv4, dv5, dv6, dv7)
    rows = (rows0, rows1, rows2, rows3)
    sem_i = (si0, si1, si2, si3, si4, si5, si6, si7)
    sem_g = (sg0, sg1, sg2, sg3)
    sem_s = (ss0, ss1, ss2, ss3)

    def chunk_off(s):
      return (wid + s * _NW) * _CH

    def issue_idx(b, off):
      pltpu.async_copy(src_hbm.at[pl.ds(off, _CH)], srcs[b], sem_i[b])
      pltpu.async_copy(dst_hbm.at[pl.ds(off, _CH)], dsts[b], sem_i[b])

    def wait_idx(b):
      pltpu.make_async_copy(
          src_hbm.at[pl.ds(0, _CH)], srcs[b], sem_i[b]).wait()
      pltpu.make_async_copy(
          dst_hbm.at[pl.ds(0, _CH)], dsts[b], sem_i[b]).wait()

    def issue_gather(b, j):
      pltpu.async_copy(feats_hbm.at[srcs[j]], rows[b], sem_g[b])

    def wait_gather(b, j):
      pltpu.make_async_copy(
          feats_hbm.at[srcs[j]], rows[b], sem_g[b]).wait()

    def issue_scatter(b, j):
      pltpu.async_copy(rows[b], acc_sh.at[dsts[j]], sem_s[b], add=True)

    def wait_scatter(b, j):
      pltpu.make_async_copy(
          rows[b], acc_sh.at[dsts[j]], sem_s[b]).wait()

    for j in range(4):
      issue_idx(j, chunk_off(j))

    z16 = jnp.zeros((16,), jnp.float32)
    for r in range(_ZR):
      for c in range(_D // 16):
        zero_v[r, pl.ds(c * 16, 16)] = z16
    for i in range(rows_per_tile // _ZR):
      pltpu.async_copy(
          zero_v, acc_sh.at[pl.ds(sid * rows_per_tile + i * _ZR, _ZR)], sz)
    for i in range(rows_per_tile // _ZR):
      pltpu.make_async_copy(
          zero_v, acc_sh.at[pl.ds(sid * rows_per_tile, _ZR)], sz).wait()
    plsc.subcore_barrier()

    def half(s, b, j, first=False, last=False):
      bn, jn = (b + 1) % 4, (j + 1) % 8
      wait_gather(b, j)
      if not first:
        wait_scatter(bn, (j + 5) % 8)
      if not last:
        wait_idx(jn)
        issue_gather(bn, jn)
      issue_scatter(b, j)
      return None

    wait_idx(0)
    issue_gather(0, 0)

    for s in range(3):
      half(s, s % 4, s % 8, first=True)
      issue_idx(s + 4, chunk_off(s + 4))
    half(3, 3, 3)
    issue_idx(7, chunk_off(7))

    def octet(k, carry):
      s0 = 8 * k + 4
      for d in range(8):
        s = s0 + d
        half(s, d % 4, (4 + d) % 8)
        issue_idx(d % 8, chunk_off(s + 4))
      return carry
    n_oct = (_CHW - 8) // 8
    lax.fori_loop(0, n_oct, octet, 0)

    for s in range(8 * n_oct + 4, _CHW - 1):
      half(s, s % 4, s % 8)
    half(_CHW - 1, (_CHW - 1) % 4, (_CHW - 1) % 8, last=True)

    for s in range(_CHW - 3, _CHW):
      wait_scatter(s % 4, s % 8)

    plsc.subcore_barrier()

    pltpu.sync_copy(
        acc_sh.at[pl.ds(sid * rows_per_tile, rows_per_tile)],
        out_hbm.at[cid, pl.ds(sid * rows_per_tile, rows_per_tile)])

  return agg


_agg = _make_agg()

_R = 1000


def _gru(parts, feats, wc_t, bc, wh_t, bh, w0_t=None):
  def body(p_ref, f_ref, wc_ref, bc_ref, wh_ref, bh_ref, *rest):
    o_ref = rest[-1]
    agg = p_ref[0] + p_ref[1]
    f = f_ref[...]
    if w0_t is not None:
      f = jnp.dot(f, rest[0][...], preferred_element_type=jnp.float32)
    gi = jnp.dot(agg, wc_ref[...], preferred_element_type=jnp.float32) + bc_ref[...]
    gh = jnp.dot(f, wh_ref[...], preferred_element_type=jnp.float32) + bh_ref[...]
    r = jax.nn.sigmoid(gi[:, :_D] + gh[:, :_D])
    z = jax.nn.sigmoid(gi[:, _D:2 * _D] + gh[:, _D:2 * _D])
    n = jnp.tanh(gi[:, 2 * _D:] + r * gh[:, 2 * _D:])
    o_ref[...] = (1.0 - z) * n + z * f

  in_specs = [
      pl.BlockSpec((_NC, _R, _D), lambda i: (0, i, 0)),
      pl.BlockSpec((_R, _D), lambda i: (i, 0)),
      pl.BlockSpec((_D, 3 * _D), lambda i: (0, 0)),
      pl.BlockSpec((1, 3 * _D), lambda i: (0, 0)),
      pl.BlockSpec((_D, 3 * _D), lambda i: (0, 0)),
      pl.BlockSpec((1, 3 * _D), lambda i: (0, 0)),
  ]
  args = [parts, feats, wc_t, bc, wh_t, bh]
  if w0_t is not None:
    in_specs.append(pl.BlockSpec((_D, _D), lambda i: (0, 0)))
    args.append(w0_t)
  return pl.pallas_call(
      body,
      grid=(_N // _R,),
      in_specs=in_specs,
      out_specs=pl.BlockSpec((_R, _D), lambda i: (i, 0)),
      out_shape=jax.ShapeDtypeStruct((_N, _D), jnp.float32),
  )(*args)


def kernel(x, edge_index, batch_size, W_init,
           W_lin0, b_lin0, W_ih0, W_hh0, b_ih0, b_hh0,
           W_lin1, b_lin1, W_ih1, W_hh1, b_ih1, b_hh1):
  src = edge_index[0].astype(jnp.int32)
  dst = edge_index[1].astype(jnp.int32)

  npadgap = _NPAD - _N
  pad = _EPAD - _E
  pad_i = jnp.arange(pad, dtype=jnp.int32)
  src_p = jnp.concatenate([src, pad_i % _N])
  dst_p = jnp.concatenate([dst, _N + pad_i % npadgap])

  wc0 = W_init.T @ W_lin0.T @ W_ih0.T
  bc0 = (b_lin0 @ W_ih0.T + b_ih0).reshape(1, -1)
  wc1 = W_lin1.T @ W_ih1.T
  bc1 = (b_lin1 @ W_ih1.T + b_ih1).reshape(1, -1)

  parts = _agg(x, src_p, dst_p)
  feats = _gru(parts, x, wc0, bc0, W_hh0.T, b_hh0.reshape(1, -1),
               w0_t=W_init.T)

  parts = _agg(feats, src_p, dst_p)
  feats = _gru(parts, feats, wc1, bc1, W_hh1.T, b_hh1.reshape(1, -1))

  return feats.reshape(_BATCH, -1, _D)
